# baseline (speedup 1.0000x reference)
import jax
import jax.numpy as jnp
from jax import lax
from jax.experimental import pallas as pl
from jax.experimental.pallas import tpu as pltpu
from jax.experimental.pallas import tpu_sc as plsc

N = 10000
E = 320000
D = 128

NC = 2
NS = 16
NW = NC * NS
EPW = E // NW
CHUNK = 40
ITERS = EPW // CHUNK
NPAD = 10240
ROWS_PER_TILE = NPAD // NS
LAST_TILE_ROWS = N - (NS - 1) * ROWS_PER_TILE


NBUF = 5
LEAD = 3


def _sc_agg_body(x_hbm, edge_hbm, parts_hbm, agg_sh, sidx_all,
                 didx0, didx1, didx2, didx3, didx4,
                 rows0, rows1, rows2, rows3, rows4,
                 semg0, semg1, semg2, semg3, semg4,
                 semi0, semi1, semi2, semi3, semi4,
                 sems0, sems1, sems2, sems3, sems4):
    didxs = (didx0, didx1, didx2, didx3, didx4)
    rows = (rows0, rows1, rows2, rows3, rows4)
    semgs = (semg0, semg1, semg2, semg3, semg4)
    semis = (semi0, semi1, semi2, semi3, semi4)
    semss = (sems0, sems1, sems2, sems3, sems4)
    c = lax.axis_index("c")
    s = lax.axis_index("s")
    w = c * NS + s
    rbase = s * ROWS_PER_TILE

    @pl.when(s < NS - 1)
    def _init_full():
        pltpu.sync_copy(x_hbm.at[pl.ds(rbase, ROWS_PER_TILE)],
                        agg_sh.at[pl.ds(rbase, ROWS_PER_TILE)])

    @pl.when(s == NS - 1)
    def _init_last():
        pltpu.sync_copy(x_hbm.at[pl.ds((NS - 1) * ROWS_PER_TILE,
                                       LAST_TILE_ROWS)],
                        agg_sh.at[pl.ds((NS - 1) * ROWS_PER_TILE,
                                        LAST_TILE_ROWS)])

    tile_base = w * EPW
    pltpu.sync_copy(edge_hbm.at[pl.ds(pl.multiple_of(tile_base, 8), EPW)],
                    sidx_all)
    plsc.subcore_barrier()

    def didx_load(i, dbuf, sem):
        ebase = pl.multiple_of(tile_base + i * CHUNK, 8)
        return pltpu.make_async_copy(edge_hbm.at[pl.ds(E + ebase, CHUNK)],
                                     dbuf, sem)

    def gather(i, buf, sem):
        sl = sidx_all.at[pl.ds(i * CHUNK, CHUNK)]
        return pltpu.make_async_copy(x_hbm.at[sl], buf, sem)

    def scat(q):
        return pltpu.make_async_copy(rows[q], agg_sh.at[didxs[q]], semss[q])

    for q in range(LEAD):
        didx_load(q, didxs[q], semis[q]).start()
        gather(q, rows[q], semgs[q]).start()

    def step(p, carry):
        i0 = NBUF * p
        for q in range(NBUF):
            i = i0 + q
            gather(i, rows[q], semgs[q]).wait()
            didx_load(i, didxs[q], semis[q]).wait()
            qf = (q + NBUF - 2) % NBUF

            @pl.when(i >= 2)
            def _drain():
                scat(qf).wait()

            scat(q).start(add=True)
            nxt = jnp.minimum(i + LEAD, ITERS - 1)
            gather(nxt, rows[qf], semgs[qf]).start()
            didx_load(nxt, didxs[qf], semis[qf]).start()
        return carry

    lax.fori_loop(0, ITERS // NBUF, step, 0)
    for i in (ITERS - 2, ITERS - 1):
        scat(i % NBUF).wait()
    for i in (ITERS, ITERS + 1, ITERS + 2):
        q = i % NBUF
        gather(ITERS - 1, rows[q], semgs[q]).wait()
        didx_load(ITERS - 1, didxs[q], semis[q]).wait()

    plsc.subcore_barrier()

    @pl.when(s < NS - 1)
    def _exp_full():
        pltpu.sync_copy(agg_sh.at[pl.ds(rbase, ROWS_PER_TILE)],
                        parts_hbm.at[c, pl.ds(rbase, ROWS_PER_TILE)])

    @pl.when(s == NS - 1)
    def _exp_last():
        pltpu.sync_copy(agg_sh.at[pl.ds((NS - 1) * ROWS_PER_TILE,
                                        LAST_TILE_ROWS)],
                        parts_hbm.at[c, pl.ds((NS - 1) * ROWS_PER_TILE,
                                              LAST_TILE_ROWS)])


_sc_agg = pl.kernel(
    _sc_agg_body,
    out_type=jax.ShapeDtypeStruct((NC, NPAD, D), jnp.float32),
    mesh=plsc.VectorSubcoreMesh(core_axis_name="c", subcore_axis_name="s"),
    scratch_types=[
        pltpu.VMEM_SHARED((NPAD, D), jnp.float32),
        pltpu.VMEM((EPW,), jnp.int32),
    ] + [pltpu.VMEM((CHUNK,), jnp.int32)] * 5
      + [pltpu.VMEM((CHUNK, D), jnp.float32)] * 5
      + [pltpu.SemaphoreType.DMA] * 15,
)


BLK = 2000
NB = N // BLK


def _tc_mlp_bn_body(parts, x, W1, W2, b1, b2, gamma, beta, y, h_s, stat_s):
    p = pl.program_id(0)
    j = pl.program_id(1)

    @pl.when(p == 0)
    def _phase0():
        hin = parts[0] + parts[1] - x[...]
        m = jnp.dot(hin, W1[...], preferred_element_type=jnp.float32)
        m = jnp.maximum(m + b1[...], 0.0)
        h = jnp.dot(m, W2[...], preferred_element_type=jnp.float32)
        h = jnp.maximum(h + b2[...], 0.0)
        h_s[pl.ds(j * BLK, BLK), :] = h
        bs = jnp.sum(h, axis=0, keepdims=True)
        bq = jnp.sum(h * h, axis=0, keepdims=True)

        @pl.when(j == 0)
        def _init():
            stat_s[0:1, :] = bs
            stat_s[1:2, :] = bq

        @pl.when(j > 0)
        def _acc():
            stat_s[0:1, :] += bs
            stat_s[1:2, :] += bq

        y[...] = h

    @pl.when(p == 1)
    def _phase1():
        mean = stat_s[0:1, :] * (1.0 / N)
        var = stat_s[1:2, :] * (1.0 / N) - mean * mean
        rstd = lax.rsqrt(var + 1e-5)
        h = h_s[pl.ds(j * BLK, BLK), :]
        y[...] = (h - mean) * (rstd * gamma[...]) + beta[...]


def _tc_mlp_bn(parts, x, W1, W2, b1, b2, gamma, beta):
    row_spec = pl.BlockSpec((BLK, D), lambda p, j: (j, 0))
    out_spec = pl.BlockSpec((BLK, D), lambda p, j: (p * j, 0))
    mat_spec = pl.BlockSpec((D, D), lambda p, j: (0, 0))
    vec_spec = pl.BlockSpec((1, D), lambda p, j: (0, 0))
    parts_spec = pl.BlockSpec((NC, BLK, D), lambda p, j: (0, j, 0))
    return pl.pallas_call(
        _tc_mlp_bn_body,
        grid=(2, NB),
        in_specs=[parts_spec, row_spec, mat_spec, mat_spec,
                  vec_spec, vec_spec, vec_spec, vec_spec],
        out_specs=out_spec,
        out_shape=jax.ShapeDtypeStruct((N, D), jnp.float32),
        scratch_shapes=[
            pltpu.VMEM((N, D), jnp.float32),
            pltpu.VMEM((2, D), jnp.float32),
        ],
        compiler_params=pltpu.CompilerParams(
            dimension_semantics=("arbitrary", "arbitrary")),
    )(parts, x, W1, W2, b1, b2, gamma, beta)


def kernel(x, edge_index, batch, W1, b1, W2, b2, gamma, beta):
    parts = _sc_agg(x, edge_index.astype(jnp.int32).reshape(2 * E))
    return _tc_mlp_bn(parts, x, W1, W2,
                      b1.reshape(1, D), b2.reshape(1, D),
                      gamma.reshape(1, D), beta.reshape(1, D))


# speedup vs baseline: 1.1280x; 1.1280x over previous
import jax
import jax.numpy as jnp
from jax import lax
from jax.experimental import pallas as pl
from jax.experimental.pallas import tpu as pltpu
from jax.experimental.pallas import tpu_sc as plsc

N = 10000
E = 320000
D = 128

NC = 2
NS = 16
NW = NC * NS
EPW = E // NW
CHUNK = 40
ITERS = EPW // CHUNK
NPAD = 10240
ROWS_PER_TILE = NPAD // NS
LAST_TILE_ROWS = N - (NS - 1) * ROWS_PER_TILE


NBUF = 7


def _sc_agg_body(x_hbm, edge_hbm, parts_hbm, agg_sh, sidx_all,
                 didx0, didx1, didx2, didx3, didx4, didx5, didx6,
                 rows0, rows1, rows2, rows3, rows4, rows5, rows6,
                 semg0, semg1, semg2, semg3, semg4, semg5, semg6,
                 semi0, semi1, semi2, semi3, semi4, semi5, semi6):
    didxs = (didx0, didx1, didx2, didx3, didx4, didx5, didx6)
    rows = (rows0, rows1, rows2, rows3, rows4, rows5, rows6)
    semgs = (semg0, semg1, semg2, semg3, semg4, semg5, semg6)
    semis = (semi0, semi1, semi2, semi3, semi4, semi5, semi6)
    c = lax.axis_index("c")
    s = lax.axis_index("s")
    w = c * NS + s
    rbase = s * ROWS_PER_TILE

    @pl.when(s < NS - 1)
    def _init_full():
        pltpu.sync_copy(x_hbm.at[pl.ds(rbase, ROWS_PER_TILE)],
                        agg_sh.at[pl.ds(rbase, ROWS_PER_TILE)])

    @pl.when(s == NS - 1)
    def _init_last():
        pltpu.sync_copy(x_hbm.at[pl.ds((NS - 1) * ROWS_PER_TILE,
                                       LAST_TILE_ROWS)],
                        agg_sh.at[pl.ds((NS - 1) * ROWS_PER_TILE,
                                        LAST_TILE_ROWS)])

    tile_base = w * EPW
    pltpu.sync_copy(edge_hbm.at[pl.ds(pl.multiple_of(tile_base, 8), EPW)],
                    sidx_all)
    plsc.subcore_barrier()

    def didx_load(i, dbuf, sem):
        ebase = pl.multiple_of(tile_base + i * CHUNK, 8)
        return pltpu.make_async_copy(edge_hbm.at[pl.ds(E + ebase, CHUNK)],
                                     dbuf, sem)

    def gather(i, buf, sem):
        sl = sidx_all.at[pl.ds(i * CHUNK, CHUNK)]
        return pltpu.make_async_copy(x_hbm.at[sl], buf, sem)

    def scatter(dbuf, buf):
        pltpu.sync_copy(buf, agg_sh.at[dbuf], add=True)

    for q in range(NBUF):
        didx_load(q, didxs[q], semis[q]).start()
        gather(q, rows[q], semgs[q]).start()

    def quad(p, carry):
        i0 = NBUF * p
        for q in range(NBUF):
            i = i0 + q
            gather(i, rows[q], semgs[q]).wait()
            didx_load(i, didxs[q], semis[q]).wait()
            scatter(didxs[q], rows[q])
            nxt = jnp.minimum(i + NBUF, ITERS - 1)
            gather(nxt, rows[q], semgs[q]).start()
            didx_load(nxt, didxs[q], semis[q]).start()
        return carry

    lax.fori_loop(0, (ITERS - 1) // NBUF, quad, 0)
    done = ((ITERS - 1) // NBUF) * NBUF
    for q in range(NBUF):
        i = min(done + q, ITERS - 1)
        gather(i, rows[q], semgs[q]).wait()
        didx_load(i, didxs[q], semis[q]).wait()
        if done + q < ITERS:
            scatter(didxs[q], rows[q])

    plsc.subcore_barrier()

    @pl.when(s < NS - 1)
    def _exp_full():
        pltpu.sync_copy(agg_sh.at[pl.ds(rbase, ROWS_PER_TILE)],
                        parts_hbm.at[c, pl.ds(rbase, ROWS_PER_TILE)])

    @pl.when(s == NS - 1)
    def _exp_last():
        pltpu.sync_copy(agg_sh.at[pl.ds((NS - 1) * ROWS_PER_TILE,
                                        LAST_TILE_ROWS)],
                        parts_hbm.at[c, pl.ds((NS - 1) * ROWS_PER_TILE,
                                              LAST_TILE_ROWS)])


_sc_agg = pl.kernel(
    _sc_agg_body,
    out_type=jax.ShapeDtypeStruct((NC, NPAD, D), jnp.float32),
    mesh=plsc.VectorSubcoreMesh(core_axis_name="c", subcore_axis_name="s"),
    scratch_types=[
        pltpu.VMEM_SHARED((NPAD, D), jnp.float32),
        pltpu.VMEM((EPW,), jnp.int32),
    ] + [pltpu.VMEM((CHUNK,), jnp.int32)] * 7
      + [pltpu.VMEM((CHUNK, D), jnp.float32)] * 7
      + [pltpu.SemaphoreType.DMA] * 14,
)


BLK = 2000
NB = N // BLK


def _tc_mlp_bn_body(parts, x, W1, W2, b1, b2, gamma, beta, y, h_s, stat_s):
    p = pl.program_id(0)
    j = pl.program_id(1)

    @pl.when(p == 0)
    def _phase0():
        hin = parts[0] + parts[1] - x[...]
        m = jnp.dot(hin, W1[...], preferred_element_type=jnp.float32)
        m = jnp.maximum(m + b1[...], 0.0)
        h = jnp.dot(m, W2[...], preferred_element_type=jnp.float32)
        h = jnp.maximum(h + b2[...], 0.0)
        h_s[pl.ds(j * BLK, BLK), :] = h
        bs = jnp.sum(h, axis=0, keepdims=True)
        bq = jnp.sum(h * h, axis=0, keepdims=True)

        @pl.when(j == 0)
        def _init():
            stat_s[0:1, :] = bs
            stat_s[1:2, :] = bq

        @pl.when(j > 0)
        def _acc():
            stat_s[0:1, :] += bs
            stat_s[1:2, :] += bq

        y[...] = h

    @pl.when(p == 1)
    def _phase1():
        mean = stat_s[0:1, :] * (1.0 / N)
        var = stat_s[1:2, :] * (1.0 / N) - mean * mean
        rstd = lax.rsqrt(var + 1e-5)
        h = h_s[pl.ds(j * BLK, BLK), :]
        y[...] = (h - mean) * (rstd * gamma[...]) + beta[...]


def _tc_mlp_bn(parts, x, W1, W2, b1, b2, gamma, beta):
    row_spec = pl.BlockSpec((BLK, D), lambda p, j: (j, 0))
    out_spec = pl.BlockSpec((BLK, D), lambda p, j: (p * j, 0))
    mat_spec = pl.BlockSpec((D, D), lambda p, j: (0, 0))
    vec_spec = pl.BlockSpec((1, D), lambda p, j: (0, 0))
    parts_spec = pl.BlockSpec((NC, BLK, D), lambda p, j: (0, j, 0))
    return pl.pallas_call(
        _tc_mlp_bn_body,
        grid=(2, NB),
        in_specs=[parts_spec, row_spec, mat_spec, mat_spec,
                  vec_spec, vec_spec, vec_spec, vec_spec],
        out_specs=out_spec,
        out_shape=jax.ShapeDtypeStruct((N, D), jnp.float32),
        scratch_shapes=[
            pltpu.VMEM((N, D), jnp.float32),
            pltpu.VMEM((2, D), jnp.float32),
        ],
        compiler_params=pltpu.CompilerParams(
            dimension_semantics=("arbitrary", "arbitrary")),
    )(parts, x, W1, W2, b1, b2, gamma, beta)


def kernel(x, edge_index, batch, W1, b1, W2, b2, gamma, beta):
    parts = _sc_agg(x, edge_index.astype(jnp.int32).reshape(2 * E))
    return _tc_mlp_bn(parts, x, W1, W2,
                      b1.reshape(1, D), b2.reshape(1, D),
                      gamma.reshape(1, D), beta.reshape(1, D))
